# baseline (speedup 1.0000x reference)
import functools

import jax
import jax.numpy as jnp
from jax import lax
from jax.experimental import pallas as pl
from jax.experimental.pallas import tpu as pltpu
from jax.experimental.pallas import tpu_sc as plsc

NDIM = 32
CTX = 20
NW = 32
PACK = 4
IW = 64


def _sc_kernel(batch):
    bpw = batch // NW
    cb = 16
    sc_chunks = bpw // cb
    rows = cb * CTX
    gi = rows // IW
    idx_rows = bpw * CTX // IW
    off_rows = sc_chunks * CTX

    mesh = plsc.VectorSubcoreMesh(core_axis_name="c", subcore_axis_name="s")

    @functools.partial(
        pl.kernel,
        mesh=mesh,
        out_type=jax.ShapeDtypeStruct((batch,), jnp.float32),
        compiler_params=pltpu.CompilerParams(
            needs_layout_passes=False, use_tc_tiling_on_sc=False),
        scratch_types=[
            pltpu.VMEM((idx_rows, IW), jnp.int32),
            pltpu.VMEM((off_rows, 16), jnp.int32),
            pltpu.VMEM((sc_chunks, 16), jnp.int32),
            pltpu.VMEM((sc_chunks, 16), jnp.int32),
            pltpu.VMEM((rows, PACK * NDIM), jnp.float32),
            pltpu.VMEM((rows, PACK * NDIM), jnp.float32),
            pltpu.VMEM((cb, PACK * NDIM), jnp.float32),
            pltpu.VMEM((cb, PACK * NDIM), jnp.float32),
            pltpu.VMEM((bpw,), jnp.float32),
            pltpu.SemaphoreType.DMA,
            pltpu.SemaphoreType.DMA,
        ],
    )
    def body(xq2d, off2d, tq2d, toff2d, embed4, embed_out4, out,
             xq_v, off_v, tq_v, toff_v, buf_a, buf_b, tbuf_a, tbuf_b,
             out_v, sem_a, sem_b):
        wid = lax.axis_index("s") * 2 + lax.axis_index("c")
        base = wid * bpw

        pltpu.sync_copy(xq2d.at[pl.ds(wid * idx_rows, idx_rows)], xq_v)
        pltpu.sync_copy(off2d.at[pl.ds(wid * off_rows, off_rows)], off_v)
        pltpu.sync_copy(tq2d.at[pl.ds(wid * sc_chunks, sc_chunks)], tq_v)
        pltpu.sync_copy(toff2d.at[pl.ds(wid * sc_chunks, sc_chunks)], toff_v)

        bufs = (buf_a, buf_b)
        tbufs = (tbuf_a, tbuf_b)
        sems = (sem_a, sem_b)

        def fire(s):
            dmas = []
            buf = bufs[s % 2]
            sem = sems[s % 2]
            for g in range(gi):
                dmas.append(pltpu.async_copy(
                    embed4.at[xq_v.at[s * gi + g]],
                    buf.at[pl.ds(g * IW, IW)], sem))
            dmas.append(pltpu.async_copy(
                embed_out4.at[tq_v.at[s]], tbufs[s % 2], sem))
            return dmas

        inflight = fire(0)
        lanes = lax.iota(jnp.int32, 16)

        for s in range(sc_chunks):
            nxt = fire(s + 1) if s + 1 < sc_chunks else []
            for d in inflight:
                d.wait()
            inflight = nxt
            buf = bufs[s % 2]
            tbuf = tbufs[s % 2]

            offs = [off_v[s * CTX + j, pl.ds(0, 16)] for j in range(CTX)]
            toffv = toff_v[s, pl.ds(0, 16)]
            rowv = lanes * CTX

            def dot_dim(d, acc):
                pool = plsc.load_gather(buf, [rowv, offs[0] + d])
                for j in range(1, CTX):
                    pool = pool + plsc.load_gather(
                        buf, [rowv + j, offs[j] + d])
                tgt = plsc.load_gather(tbuf, [lanes, toffv + d])
                return acc + pool * tgt
            acc = lax.fori_loop(0, NDIM, dot_dim, lanes * jnp.float32(0),
                                unroll=False)
            out_v[pl.ds(s * cb, cb)] = acc

        pltpu.sync_copy(out_v, out.at[pl.ds(base, bpw)])

    return body


def kernel(x, target_id, embed, embed_out):
    batch, ctx = x.shape
    assert ctx == CTX
    xq2d = lax.shift_right_logical(x, 2).reshape(batch * CTX // IW, IW)
    off = ((x & 3) * NDIM).reshape(batch // 16, 16, CTX)
    off2d = off.transpose(0, 2, 1).reshape(batch // 16 * CTX, 16)
    tq2d = lax.shift_right_logical(target_id, 2).reshape(batch // 16, 16)
    toff2d = ((target_id & 3) * NDIM).reshape(batch // 16, 16)
    embed4 = embed.reshape(-1, PACK * NDIM)
    embed_out4 = embed_out.reshape(-1, PACK * NDIM)
    return _sc_kernel(batch)(xq2d, off2d, tq2d, toff2d, embed4, embed_out4)
